# baseline (speedup 1.0000x reference)
import jax
import jax.numpy as jnp
from jax import lax
from jax.experimental import pallas as pl
from jax.experimental.pallas import tpu as pltpu
from jax.experimental.pallas import tpu_sc as plsc

N_USERS = 100000
N_ITEMS = 100000
N_F = 32
ITEM_BIN = 30
MAXDAY = 3000
HIST = 20
B = 16384
BETA = 0.4

NC = 2
NS = 16
NW = NC * NS
BPW = B // NW
NCH = BPW // 128
NG = BPW // 16
ROWS_PER_G = 3
GSLOT = ROWS_PER_G * 128

R16 = N_USERS // 16
B_BU = 0 * R16
B_MUD = 1 * R16
B_AL = 2 * R16
B_BCU = 3 * R16
B_UIC = 4 * R16
B_BI = 5 * R16
B_WBIT = 6 * R16
B_URI = 6 * R16 + N_ITEMS * ITEM_BIN // 16

LN2 = 0.6931471805599453


def _pow_approx(a, p):
  bits = plsc.bitcast(a, jnp.int32)
  e = (bits >> 23) - 127
  m = plsc.bitcast((bits & 0x007FFFFF) | 0x3F800000, jnp.float32)
  t = (m - 1.0) / (m + 1.0)
  t2 = t * t
  lnm = 2.0 * t * (1.0 + t2 * (1.0 / 3.0 + t2 * (1.0 / 5.0 + t2 * (1.0 / 7.0 + t2 * (1.0 / 9.0)))))
  ln = e.astype(jnp.float32) * LN2 + lnm
  return jnp.exp(p * ln)


def _body(user_h, item_h, tbin_h, tday_h, mc_h, aux_h, btd_h, wcu_h,
          wpu_h, wpi_h, auk_h, pkut_h, y_h, lr_h, gm_h,
          out_h,
          u1, it1, tb1, mc1, wb1, td_v,
          xbu, xmud, xal, xbcu, xuic, xbi, xwbit, xua, xub,
          bu_v, mud_v, al_v, bcu_v, uic_v, bi_v, wbit_v, uriA, uriB,
          ring0, ring1, ring2, ring3,
          btd_v, wcu_v, jflat, cnt_v,
          wrow0, wrow1, irow0, irow1, arow0, arow1, prow0, prow1,
          ybuf0, ybuf1, lr_v, gm_v, out_v,
          s_tab, s_p0, s_p1, r0, r1, r2, r3):
  wid = lax.axis_index("s") * NC + lax.axis_index("c")
  base = wid * BPW
  iota = lax.iota(jnp.int32, 16)

  h0 = []
  bsl = pl.ds(base, BPW)
  h0.append(pltpu.async_copy(user_h.at[bsl], u1, s_tab))
  h0.append(pltpu.async_copy(item_h.at[bsl], it1, s_tab))
  h0.append(pltpu.async_copy(tbin_h.at[bsl], tb1, s_tab))
  h0.append(pltpu.async_copy(tday_h.at[bsl], td_v, s_tab))
  h0.append(pltpu.async_copy(mc_h.at[bsl], mc1, s_tab))
  h0.append(pltpu.async_copy(btd_h, btd_v, s_tab))
  h0.append(pltpu.async_copy(wcu_h, wcu_v, s_tab))
  h0.append(pltpu.async_copy(lr_h, lr_v, s_tab))
  h0.append(pltpu.async_copy(gm_h, gm_v, s_tab))
  for h in h0:
    h.wait()

  for k in range(BPW // 16):
    sl = pl.ds(k * 16, 16)
    u = u1[sl]
    it = it1[sl]
    wb = it * ITEM_BIN + tb1[sl]
    wb1[sl] = wb
    ur = u >> 4
    xbu[sl] = ur + B_BU
    xmud[sl] = ur + B_MUD
    xal[sl] = ur + B_AL
    xbcu[sl] = ur + B_BCU
    xuic[sl] = ur + B_UIC
    xbi[sl] = (it >> 4) + B_BI
    xwbit[sl] = (wb >> 4) + B_WBIT
    ua = u + (u >> 2) + B_URI
    xua[sl] = ua
    xub[sl] = ua + 1

  hu = []
  for c in range(NCH):
    csl = pl.ds(c * 128, 128)
    hu.append(pltpu.async_copy(aux_h.at[xua.at[csl]], uriA.at[csl], s_p0))
    hu.append(pltpu.async_copy(aux_h.at[xub.at[csl]], uriB.at[csl], s_p0))

  jobs = []
  for c in range(NCH):
    for xb, colb, dst in ((xbu, u1, bu_v), (xmud, u1, mud_v),
                          (xal, u1, al_v), (xbcu, u1, bcu_v),
                          (xuic, u1, uic_v), (xbi, it1, bi_v),
                          (xwbit, wb1, wbit_v)):
      jobs.append((xb, colb, dst, c))
  rings = (ring0, ring1, ring2, ring3)
  rsems = (r0, r1, r2, r3)

  def job_copy(j):
    xb, colb, dst, c = jobs[j]
    return pltpu.make_async_copy(aux_h.at[xb.at[pl.ds(c * 128, 128)]],
                                 rings[j % 4], rsems[j % 4])

  for j in range(4):
    job_copy(j).start()
  for j in range(len(jobs)):
    job_copy(j).wait()
    xb, colb, dst, c = jobs[j]
    ring = rings[j % 4]
    for k in range(8):
      sl = pl.ds(c * 128 + k * 16, 16)
      dst[sl] = plsc.load_gather(ring, [k * 16 + iota, colb[sl] & 15])
    if j + 4 < len(jobs):
      job_copy(j + 4).start()
  for h in hu:
    h.wait()

  def clamp_body(g, _):
    bl = g * 16 + iota
    u16 = u1[pl.ds(pl.multiple_of(g * 16, 16), 16)]
    off = (u16 & 3) * 4
    cnt = jnp.zeros((16,), jnp.float32)
    gslot = g * GSLOT
    for h in range(HIST):
      pos = off + h
      lane = pos & 15
      jA = plsc.bitcast(plsc.load_gather(uriA, [bl, lane]), jnp.int32)
      jB = plsc.bitcast(plsc.load_gather(uriB, [bl, lane]), jnp.int32)
      j = jnp.where(pos < 16, jA, jB)
      cnt = cnt + jnp.where(j == N_ITEMS, 1.0, 0.0)
      jc = jnp.minimum(j, N_ITEMS - 1)
      slot = gslot + iota * HIST + h
      plsc.store_scatter(jflat, [slot >> 7, slot & 127], jc)
    cnt_v[pl.ds(pl.multiple_of(g * 16, 16), 16)] = cnt
    for k in range(4):
      pslot = gslot + 320 + k * 16 + iota
      pval = ((g * 64 + k * 16 + iota) * 32 + wid) & 0xFFFF
      plsc.store_scatter(jflat, [pslot >> 7, pslot & 127], pval)
    return 0

  lax.fori_loop(0, NG, clamp_body, 0)

  def g_copies(g, bufs, sem):
    wrow, irow, arow, prow, ybuf = bufs
    gsl = pl.ds(pl.multiple_of(g * 16, 16), 16)
    cps = [
        pltpu.make_async_copy(wpu_h.at[u1.at[gsl]], wrow, sem),
        pltpu.make_async_copy(wpi_h.at[it1.at[gsl]], irow, sem),
        pltpu.make_async_copy(auk_h.at[u1.at[gsl]], arow, sem),
        pltpu.make_async_copy(pkut_h.at[mc1.at[gsl]], prow, sem),
    ]
    for c in range(ROWS_PER_G):
      cps.append(pltpu.make_async_copy(y_h.at[jflat.at[ROWS_PER_G * g + c]],
                                       ybuf.at[pl.ds(c * 128, 128)], sem))
    return cps

  def fire_g(g, bufs, sem):
    for cp in g_copies(g, bufs, sem):
      cp.start()

  def drain_g(g, bufs, sem):
    for cp in g_copies(g, bufs, sem):
      cp.wait()

  bufs0 = (wrow0, irow0, arow0, prow0, ybuf0)
  bufs1 = (wrow1, irow1, arow1, prow1, ybuf1)

  gm16 = gm_v[pl.ds(0, 16)]
  lrA = lr_v[pl.ds(0, 16)]
  lrB = lr_v[pl.ds(16, 16)]

  def compute_group(g, bufs):
    wrow, irow, arow, prow, ybuf = bufs
    sl = pl.ds(pl.multiple_of(g * 16, 16), 16)
    bu = bu_v[sl]
    mud = mud_v[sl]
    al = al_v[sl]
    bcu = bcu_v[sl]
    uic = uic_v[sl]
    bi_ = bi_v[sl]
    wbit = wbit_v[sl]
    cnt = cnt_v[sl]
    td = td_v[sl].astype(jnp.float32)
    mc16 = mc1[sl]
    btd = plsc.load_gather(btd_v, [mc16])
    wcu = plsc.load_gather(wcu_v, [mc16])
    d = td - mud
    dev = jnp.sign(d) * _pow_approx(jnp.abs(d), BETA)
    ru = _pow_approx(uic, -0.5)
    acc = jnp.zeros((16,), jnp.float32)
    for b in range(16):
      h0 = pl.ds(0, 16)
      h1 = pl.ds(16, 16)
      s0 = ((ybuf[b * HIST + 0, h0] + ybuf[b * HIST + 1, h0])
            + (ybuf[b * HIST + 2, h0] + ybuf[b * HIST + 3, h0]))
      s1 = ((ybuf[b * HIST + 0, h1] + ybuf[b * HIST + 1, h1])
            + (ybuf[b * HIST + 2, h1] + ybuf[b * HIST + 3, h1]))
      for h in range(4, HIST, 4):
        s0 = s0 + ((ybuf[b * HIST + h, h0] + ybuf[b * HIST + h + 1, h0])
                   + (ybuf[b * HIST + h + 2, h0] + ybuf[b * HIST + h + 3, h0]))
        s1 = s1 + ((ybuf[b * HIST + h, h1] + ybuf[b * HIST + h + 1, h1])
                   + (ybuf[b * HIST + h + 2, h1] + ybuf[b * HIST + h + 3, h1]))
      cb = cnt[b]
      s0 = s0 - cb * lrA
      s1 = s1 - cb * lrB
      rb = ru[b]
      db = dev[b]
      uvt0 = wrow[b, h0] + rb * s0 + db * arow[b, h0] + prow[b, h0]
      uvt1 = wrow[b, h1] + rb * s1 + db * arow[b, h1] + prow[b, h1]
      prod = uvt0 * irow[b, h0] + uvt1 * irow[b, h1]
      acc = acc + jnp.where(iota == b, jnp.sum(prod), 0.0)
    pred = (gm16 + bu + al * dev + btd
            + (bi_ + wbit) * (bcu + wcu) + acc)
    out_v[sl] = pred

  fire_g(0, bufs0, s_p0)
  fire_g(1, bufs1, s_p1)

  def pipe_body(i, _):
    g0 = i * 2
    g1 = i * 2 + 1
    drain_g(g0, bufs0, s_p0)
    compute_group(g0, bufs0)
    fire_g(jnp.minimum(g0 + 2, NG - 1), bufs0, s_p0)
    drain_g(g1, bufs1, s_p1)
    compute_group(g1, bufs1)
    fire_g(jnp.minimum(g1 + 2, NG - 1), bufs1, s_p1)
    return 0

  lax.fori_loop(0, NG // 2, pipe_body, 0)
  drain_g(NG - 1, bufs0, s_p0)
  drain_g(NG - 1, bufs1, s_p1)

  pltpu.sync_copy(out_v, out_h.at[pl.ds(base, BPW)])


@jax.jit
def _run(user, item, tbin, tday, mc, aux, btd, wcu, wpu, wpi, auk, pkut,
         y, lr, gm):
  mesh = plsc.VectorSubcoreMesh(core_axis_name="c", subcore_axis_name="s",
                                num_cores=NC, num_subcores=NS)
  f = pl.kernel(
      _body,
      out_type=jax.ShapeDtypeStruct((B,), jnp.float32),
      mesh=mesh,
      scratch_types=[
          pltpu.VMEM((BPW,), jnp.int32),
          pltpu.VMEM((BPW,), jnp.int32),
          pltpu.VMEM((BPW,), jnp.int32),
          pltpu.VMEM((BPW,), jnp.int32),
          pltpu.VMEM((BPW,), jnp.int32),
          pltpu.VMEM((BPW,), jnp.int32),
          pltpu.VMEM((BPW,), jnp.int32),
          pltpu.VMEM((BPW,), jnp.int32),
          pltpu.VMEM((BPW,), jnp.int32),
          pltpu.VMEM((BPW,), jnp.int32),
          pltpu.VMEM((BPW,), jnp.int32),
          pltpu.VMEM((BPW,), jnp.int32),
          pltpu.VMEM((BPW,), jnp.int32),
          pltpu.VMEM((BPW,), jnp.int32),
          pltpu.VMEM((BPW,), jnp.int32),
          pltpu.VMEM((BPW,), jnp.float32),
          pltpu.VMEM((BPW,), jnp.float32),
          pltpu.VMEM((BPW,), jnp.float32),
          pltpu.VMEM((BPW,), jnp.float32),
          pltpu.VMEM((BPW,), jnp.float32),
          pltpu.VMEM((BPW,), jnp.float32),
          pltpu.VMEM((BPW,), jnp.float32),
          pltpu.VMEM((BPW, 16), jnp.float32),
          pltpu.VMEM((BPW, 16), jnp.float32),
          pltpu.VMEM((128, 16), jnp.float32),
          pltpu.VMEM((128, 16), jnp.float32),
          pltpu.VMEM((128, 16), jnp.float32),
          pltpu.VMEM((128, 16), jnp.float32),
          pltpu.VMEM((MAXDAY + 1,), jnp.float32),
          pltpu.VMEM((MAXDAY + 1,), jnp.float32),
          pltpu.VMEM((NG * ROWS_PER_G, 128), jnp.int32),
          pltpu.VMEM((BPW,), jnp.float32),
          pltpu.VMEM((16, N_F), jnp.float32),
          pltpu.VMEM((16, N_F), jnp.float32),
          pltpu.VMEM((16, N_F), jnp.float32),
          pltpu.VMEM((16, N_F), jnp.float32),
          pltpu.VMEM((16, N_F), jnp.float32),
          pltpu.VMEM((16, N_F), jnp.float32),
          pltpu.VMEM((16, N_F), jnp.float32),
          pltpu.VMEM((16, N_F), jnp.float32),
          pltpu.VMEM((GSLOT, N_F), jnp.float32),
          pltpu.VMEM((GSLOT, N_F), jnp.float32),
          pltpu.VMEM((N_F,), jnp.float32),
          pltpu.VMEM((16,), jnp.float32),
          pltpu.VMEM((BPW,), jnp.float32),
          pltpu.SemaphoreType.DMA,
          pltpu.SemaphoreType.DMA,
          pltpu.SemaphoreType.DMA,
          pltpu.SemaphoreType.DMA,
          pltpu.SemaphoreType.DMA,
          pltpu.SemaphoreType.DMA,
          pltpu.SemaphoreType.DMA,
      ],
      compiler_params=pltpu.CompilerParams(needs_layout_passes=False,
                                           use_tc_tiling_on_sc=False),
  )
  return f(user, item, tbin, tday, mc, aux, btd, wcu, wpu, wpi, auk, pkut,
           y, lr, gm)


def kernel(user, item, tbin, tday, mean_ud, global_mean, maxday_cat,
           user_itemcount, user_rated_item, WPI, WPU, BU, BI, WBIT, Alpha,
           AlphaUK, WPUKT, BTDay, BCU, WCU, Y):
  aux = jnp.concatenate([
      BU.reshape(-1, 16),
      mean_ud.reshape(-1, 16),
      Alpha.reshape(-1, 16),
      BCU.reshape(-1, 16),
      user_itemcount.astype(jnp.float32).reshape(-1, 16),
      BI.reshape(-1, 16),
      WBIT.reshape(-1, 16),
      lax.bitcast_convert_type(user_rated_item, jnp.float32).reshape(-1, 16),
  ], axis=0)
  lr32 = Y[N_ITEMS - 1]
  gm16 = jnp.broadcast_to(global_mean, (16,)).astype(jnp.float32)
  return _run(user, item, tbin, tday, maxday_cat, aux, BTDay, WCU, WPU,
              WPI, AlphaUK, WPUKT, Y, lr32, gm16)


# speedup vs baseline: 2.1965x; 2.1965x over previous
import jax
import jax.numpy as jnp
from jax import lax
from jax.experimental import pallas as pl
from jax.experimental.pallas import tpu as pltpu
from jax.experimental.pallas import tpu_sc as plsc

N_USERS = 100000
N_ITEMS = 100000
N_F = 32
ITEM_BIN = 30
MAXDAY = 3000
HIST = 20
B = 16384
BETA = 0.4

NC = 2
NS = 16
NW = NC * NS
BPW = B // NW
NCH = BPW // 128
NG = BPW // 16
ROWS_PER_G = 3
GSLOT = ROWS_PER_G * 128

R16 = N_USERS // 16
B_BU = 0 * R16
B_MUD = 1 * R16
B_AL = 2 * R16
B_BCU = 3 * R16
B_UIC = 4 * R16
B_BI = 5 * R16

LN2 = 0.6931471805599453


def _pow_approx(a, p):
  bits = plsc.bitcast(a, jnp.int32)
  e = (bits >> 23) - 127
  m = plsc.bitcast((bits & 0x007FFFFF) | 0x3F800000, jnp.float32)
  t = (m - 1.0) / (m + 1.0)
  t2 = t * t
  lnm = 2.0 * t * (1.0 + t2 * (1.0 / 3.0 + t2 * (1.0 / 5.0 + t2 * (1.0 / 7.0 + t2 * (1.0 / 9.0)))))
  ln = e.astype(jnp.float32) * LN2 + lnm
  return jnp.exp(p * ln)


def _body(user_h, item_h, tbin_h, tday_h, mc_h, pk6_h, wbit16_h, uri16_h,
          btd_h, wcu_h,
          wpu_h, wpi_h, auk_h, pkut_h, y_h, lr_h, gm_h,
          out_h,
          u1, it1, tb1, mc1, wb1, td_v,
          xbu, xmud, xal, xbcu, xuic, xbi, xwbit, xua, xub,
          bu_v, mud_v, al_v, bcu_v, uic_v, bi_v, wbit_v, uriA, uriB,
          ring0, ring1, ring2, ring3,
          btd_v, wcu_v, jflat, cnt_v,
          wrow0, wrow1, irow0, irow1, arow0, arow1, prow0, prow1,
          ybuf0, ybuf1, lr_v, gm_v, out_v,
          s_tab, s_p0, s_p1, r0, r1, r2, r3):
  wid = lax.axis_index("s") * NC + lax.axis_index("c")
  base = wid * BPW
  iota = lax.iota(jnp.int32, 16)

  h0 = []
  bsl = pl.ds(base, BPW)
  h0.append(pltpu.async_copy(user_h.at[bsl], u1, s_tab))
  h0.append(pltpu.async_copy(item_h.at[bsl], it1, s_tab))
  h0.append(pltpu.async_copy(tbin_h.at[bsl], tb1, s_tab))
  h0.append(pltpu.async_copy(tday_h.at[bsl], td_v, s_tab))
  h0.append(pltpu.async_copy(mc_h.at[bsl], mc1, s_tab))
  h0.append(pltpu.async_copy(btd_h, btd_v, s_tab))
  h0.append(pltpu.async_copy(wcu_h, wcu_v, s_tab))
  h0.append(pltpu.async_copy(lr_h, lr_v, s_tab))
  h0.append(pltpu.async_copy(gm_h, gm_v, s_tab))
  for h in h0:
    h.wait()

  for k in range(BPW // 16):
    sl = pl.ds(k * 16, 16)
    u = u1[sl]
    it = it1[sl]
    wb = it * ITEM_BIN + tb1[sl]
    wb1[sl] = wb
    ur = u >> 4
    xbu[sl] = ur + B_BU
    xmud[sl] = ur + B_MUD
    xal[sl] = ur + B_AL
    xbcu[sl] = ur + B_BCU
    xuic[sl] = ur + B_UIC
    xbi[sl] = (it >> 4) + B_BI
    xwbit[sl] = wb >> 4
    ua = u + (u >> 2)
    xua[sl] = ua
    xub[sl] = ua + 1

  hu = []
  for c in range(NCH):
    csl = pl.ds(c * 128, 128)
    hu.append(pltpu.async_copy(uri16_h.at[xua.at[csl]], uriA.at[csl], s_p0))
    hu.append(pltpu.async_copy(uri16_h.at[xub.at[csl]], uriB.at[csl], s_p0))

  jobs = []
  for c in range(NCH):
    for tab, xb, colb, dst in ((pk6_h, xbu, u1, bu_v),
                               (pk6_h, xmud, u1, mud_v),
                               (pk6_h, xal, u1, al_v),
                               (pk6_h, xbcu, u1, bcu_v),
                               (pk6_h, xuic, u1, uic_v),
                               (pk6_h, xbi, it1, bi_v),
                               (wbit16_h, xwbit, wb1, wbit_v)):
      jobs.append((tab, xb, colb, dst, c))
  rings = (ring0, ring1, ring2, ring3)
  rsems = (r0, r1, r2, r3)

  def job_copy(j):
    tab, xb, colb, dst, c = jobs[j]
    return pltpu.make_async_copy(tab.at[xb.at[pl.ds(c * 128, 128)]],
                                 rings[j % 4], rsems[j % 4])

  for j in range(4):
    job_copy(j).start()
  for j in range(len(jobs)):
    job_copy(j).wait()
    tab, xb, colb, dst, c = jobs[j]
    ring = rings[j % 4]
    for k in range(8):
      sl = pl.ds(c * 128 + k * 16, 16)
      dst[sl] = plsc.load_gather(ring, [k * 16 + iota, colb[sl] & 15])
    if j + 4 < len(jobs):
      job_copy(j + 4).start()
  for h in hu:
    h.wait()

  def clamp_body(g, _):
    bl = g * 16 + iota
    u16 = u1[pl.ds(pl.multiple_of(g * 16, 16), 16)]
    off = (u16 & 3) * 4
    cnt = jnp.zeros((16,), jnp.float32)
    gslot = g * GSLOT
    for h in range(HIST):
      pos = off + h
      lane = pos & 15
      jA = plsc.load_gather(uriA, [bl, lane])
      jB = plsc.load_gather(uriB, [bl, lane])
      j = jnp.where(pos < 16, jA, jB)
      cnt = cnt + jnp.where(j == N_ITEMS, 1.0, 0.0)
      jc = jnp.minimum(j, N_ITEMS - 1)
      slot = gslot + iota * HIST + h
      plsc.store_scatter(jflat, [slot >> 7, slot & 127], jc)
    cnt_v[pl.ds(pl.multiple_of(g * 16, 16), 16)] = cnt
    for k in range(4):
      pslot = gslot + 320 + k * 16 + iota
      pval = ((g * 64 + k * 16 + iota) * 32 + wid) & 0xFFFF
      plsc.store_scatter(jflat, [pslot >> 7, pslot & 127], pval)
    return 0

  lax.fori_loop(0, NG, clamp_body, 0)

  def g_copies(g, bufs, sem):
    wrow, irow, arow, prow, ybuf = bufs
    gsl = pl.ds(pl.multiple_of(g * 16, 16), 16)
    cps = [
        pltpu.make_async_copy(wpu_h.at[u1.at[gsl]], wrow, sem),
        pltpu.make_async_copy(wpi_h.at[it1.at[gsl]], irow, sem),
        pltpu.make_async_copy(auk_h.at[u1.at[gsl]], arow, sem),
        pltpu.make_async_copy(pkut_h.at[mc1.at[gsl]], prow, sem),
    ]
    for c in range(ROWS_PER_G):
      cps.append(pltpu.make_async_copy(y_h.at[jflat.at[ROWS_PER_G * g + c]],
                                       ybuf.at[pl.ds(c * 128, 128)], sem))
    return cps

  def fire_g(g, bufs, sem):
    for cp in g_copies(g, bufs, sem):
      cp.start()

  def drain_g(g, bufs, sem):
    for cp in g_copies(g, bufs, sem):
      cp.wait()

  bufs0 = (wrow0, irow0, arow0, prow0, ybuf0)
  bufs1 = (wrow1, irow1, arow1, prow1, ybuf1)

  gm16 = gm_v[pl.ds(0, 16)]
  lrA = lr_v[pl.ds(0, 16)]
  lrB = lr_v[pl.ds(16, 16)]

  def compute_group(g, bufs):
    wrow, irow, arow, prow, ybuf = bufs
    sl = pl.ds(pl.multiple_of(g * 16, 16), 16)
    bu = bu_v[sl]
    mud = mud_v[sl]
    al = al_v[sl]
    bcu = bcu_v[sl]
    uic = uic_v[sl]
    bi_ = bi_v[sl]
    wbit = wbit_v[sl]
    cnt = cnt_v[sl]
    td = td_v[sl].astype(jnp.float32)
    mc16 = mc1[sl]
    btd = plsc.load_gather(btd_v, [mc16])
    wcu = plsc.load_gather(wcu_v, [mc16])
    d = td - mud
    dev = jnp.sign(d) * _pow_approx(jnp.abs(d), BETA)
    ru = _pow_approx(uic, -0.5)
    acc = jnp.zeros((16,), jnp.float32)
    for b in range(16):
      h0 = pl.ds(0, 16)
      h1 = pl.ds(16, 16)
      s0 = ((ybuf[b * HIST + 0, h0] + ybuf[b * HIST + 1, h0])
            + (ybuf[b * HIST + 2, h0] + ybuf[b * HIST + 3, h0]))
      s1 = ((ybuf[b * HIST + 0, h1] + ybuf[b * HIST + 1, h1])
            + (ybuf[b * HIST + 2, h1] + ybuf[b * HIST + 3, h1]))
      for h in range(4, HIST, 4):
        s0 = s0 + ((ybuf[b * HIST + h, h0] + ybuf[b * HIST + h + 1, h0])
                   + (ybuf[b * HIST + h + 2, h0] + ybuf[b * HIST + h + 3, h0]))
        s1 = s1 + ((ybuf[b * HIST + h, h1] + ybuf[b * HIST + h + 1, h1])
                   + (ybuf[b * HIST + h + 2, h1] + ybuf[b * HIST + h + 3, h1]))
      cb = cnt[b]
      s0 = s0 - cb * lrA
      s1 = s1 - cb * lrB
      rb = ru[b]
      db = dev[b]
      uvt0 = wrow[b, h0] + rb * s0 + db * arow[b, h0] + prow[b, h0]
      uvt1 = wrow[b, h1] + rb * s1 + db * arow[b, h1] + prow[b, h1]
      prod = uvt0 * irow[b, h0] + uvt1 * irow[b, h1]
      acc = acc + jnp.where(iota == b, jnp.sum(prod), 0.0)
    pred = (gm16 + bu + al * dev + btd
            + (bi_ + wbit) * (bcu + wcu) + acc)
    out_v[sl] = pred

  fire_g(0, bufs0, s_p0)
  fire_g(1, bufs1, s_p1)

  def pipe_body(i, _):
    g0 = i * 2
    g1 = i * 2 + 1
    drain_g(g0, bufs0, s_p0)
    compute_group(g0, bufs0)
    fire_g(jnp.minimum(g0 + 2, NG - 1), bufs0, s_p0)
    drain_g(g1, bufs1, s_p1)
    compute_group(g1, bufs1)
    fire_g(jnp.minimum(g1 + 2, NG - 1), bufs1, s_p1)
    return 0

  lax.fori_loop(0, NG // 2, pipe_body, 0)
  drain_g(NG - 1, bufs0, s_p0)
  drain_g(NG - 1, bufs1, s_p1)

  pltpu.sync_copy(out_v, out_h.at[pl.ds(base, BPW)])


@jax.jit
def _run(user, item, tbin, tday, mc, pk6, wbit16, uri16, btd, wcu,
         wpu, wpi, auk, pkut, y, lr, gm):
  mesh = plsc.VectorSubcoreMesh(core_axis_name="c", subcore_axis_name="s",
                                num_cores=NC, num_subcores=NS)
  f = pl.kernel(
      _body,
      out_type=jax.ShapeDtypeStruct((B,), jnp.float32),
      mesh=mesh,
      scratch_types=[
          pltpu.VMEM((BPW,), jnp.int32),
          pltpu.VMEM((BPW,), jnp.int32),
          pltpu.VMEM((BPW,), jnp.int32),
          pltpu.VMEM((BPW,), jnp.int32),
          pltpu.VMEM((BPW,), jnp.int32),
          pltpu.VMEM((BPW,), jnp.int32),
          pltpu.VMEM((BPW,), jnp.int32),
          pltpu.VMEM((BPW,), jnp.int32),
          pltpu.VMEM((BPW,), jnp.int32),
          pltpu.VMEM((BPW,), jnp.int32),
          pltpu.VMEM((BPW,), jnp.int32),
          pltpu.VMEM((BPW,), jnp.int32),
          pltpu.VMEM((BPW,), jnp.int32),
          pltpu.VMEM((BPW,), jnp.int32),
          pltpu.VMEM((BPW,), jnp.int32),
          pltpu.VMEM((BPW,), jnp.float32),
          pltpu.VMEM((BPW,), jnp.float32),
          pltpu.VMEM((BPW,), jnp.float32),
          pltpu.VMEM((BPW,), jnp.float32),
          pltpu.VMEM((BPW,), jnp.float32),
          pltpu.VMEM((BPW,), jnp.float32),
          pltpu.VMEM((BPW,), jnp.float32),
          pltpu.VMEM((BPW, 16), jnp.int32),
          pltpu.VMEM((BPW, 16), jnp.int32),
          pltpu.VMEM((128, 16), jnp.float32),
          pltpu.VMEM((128, 16), jnp.float32),
          pltpu.VMEM((128, 16), jnp.float32),
          pltpu.VMEM((128, 16), jnp.float32),
          pltpu.VMEM((MAXDAY + 1,), jnp.float32),
          pltpu.VMEM((MAXDAY + 1,), jnp.float32),
          pltpu.VMEM((NG * ROWS_PER_G, 128), jnp.int32),
          pltpu.VMEM((BPW,), jnp.float32),
          pltpu.VMEM((16, N_F), jnp.float32),
          pltpu.VMEM((16, N_F), jnp.float32),
          pltpu.VMEM((16, N_F), jnp.float32),
          pltpu.VMEM((16, N_F), jnp.float32),
          pltpu.VMEM((16, N_F), jnp.float32),
          pltpu.VMEM((16, N_F), jnp.float32),
          pltpu.VMEM((16, N_F), jnp.float32),
          pltpu.VMEM((16, N_F), jnp.float32),
          pltpu.VMEM((GSLOT, N_F), jnp.float32),
          pltpu.VMEM((GSLOT, N_F), jnp.float32),
          pltpu.VMEM((N_F,), jnp.float32),
          pltpu.VMEM((16,), jnp.float32),
          pltpu.VMEM((BPW,), jnp.float32),
          pltpu.SemaphoreType.DMA,
          pltpu.SemaphoreType.DMA,
          pltpu.SemaphoreType.DMA,
          pltpu.SemaphoreType.DMA,
          pltpu.SemaphoreType.DMA,
          pltpu.SemaphoreType.DMA,
          pltpu.SemaphoreType.DMA,
      ],
      compiler_params=pltpu.CompilerParams(needs_layout_passes=False,
                                           use_tc_tiling_on_sc=False),
  )
  return f(user, item, tbin, tday, mc, pk6, wbit16, uri16, btd, wcu,
           wpu, wpi, auk, pkut, y, lr, gm)


def kernel(user, item, tbin, tday, mean_ud, global_mean, maxday_cat,
           user_itemcount, user_rated_item, WPI, WPU, BU, BI, WBIT, Alpha,
           AlphaUK, WPUKT, BTDay, BCU, WCU, Y):
  pk6 = jnp.concatenate([
      BU, mean_ud, Alpha, BCU,
      user_itemcount.astype(jnp.float32), BI,
  ]).reshape(-1, 16)
  wbit16 = WBIT.reshape(-1, 16)
  uri16 = user_rated_item.reshape(-1, 16)
  lr32 = Y[N_ITEMS - 1]
  gm16 = jnp.broadcast_to(global_mean, (16,)).astype(jnp.float32)
  return _run(user, item, tbin, tday, maxday_cat, pk6, wbit16, uri16,
              BTDay, WCU, WPU, WPI, AlphaUK, WPUKT, Y, lr32, gm16)


# speedup vs baseline: 2.6872x; 1.2234x over previous
import jax
import jax.numpy as jnp
from jax import lax
from jax.experimental import pallas as pl
from jax.experimental.pallas import tpu as pltpu
from jax.experimental.pallas import tpu_sc as plsc

N_USERS = 100000
N_ITEMS = 100000
N_F = 32
ITEM_BIN = 30
MAXDAY = 3000
HIST = 20
B = 16384
BETA = 0.4

NC = 2
NS = 16
NW = NC * NS
BPW = B // NW
NCH = BPW // 128
NG = BPW // 16
ROWS_PER_G = 3
GSLOT = ROWS_PER_G * 128

R16 = N_USERS // 16
B_BU = 0 * R16
B_MUD = 1 * R16
B_AL = 2 * R16
B_BCU = 3 * R16
B_UIC = 4 * R16
B_BI = 5 * R16

LN2 = 0.6931471805599453


def _pow_approx(a, p):
  bits = plsc.bitcast(a, jnp.int32)
  e = (bits >> 23) - 127
  m = plsc.bitcast((bits & 0x007FFFFF) | 0x3F800000, jnp.float32)
  t = (m - 1.0) / (m + 1.0)
  t2 = t * t
  lnm = 2.0 * t * (1.0 + t2 * (1.0 / 3.0 + t2 * (1.0 / 5.0 + t2 * (1.0 / 7.0 + t2 * (1.0 / 9.0)))))
  ln = e.astype(jnp.float32) * LN2 + lnm
  return jnp.exp(p * ln)


def _body(user_h, item_h, tbin_h, tday_h, mc_h,
          bu_h, mud_h, al_h, bcu_h, uicf_h, bi_h, wbit16_h, uri16_h,
          btd_h, wcu_h,
          wpu_h, wpi_h, auk_h, pkut_h, y_h, lr_h, gm_h,
          out_h,
          u1, it1, tb1, mc1, wb1, td_v,
          xwbit, xua, xub,
          bu_v, mud_v, al_v, bcu_v, uic_v, bi_v, wbit_v, uriA, uriB,
          ring0, ring1, ring2, ring3,
          btd_v, wcu_v, jflat, cnt_v,
          wrow0, wrow1, irow0, irow1, arow0, arow1, prow0, prow1,
          ybuf0, ybuf1, lr_v, gm_v, out_v,
          s_tab, s_p0, s_p1, r0, r1, r2, r3):
  wid = lax.axis_index("s") * NC + lax.axis_index("c")
  base = wid * BPW
  iota = lax.iota(jnp.int32, 16)

  h0 = []
  bsl = pl.ds(base, BPW)
  h0.append(pltpu.async_copy(user_h.at[bsl], u1, s_tab))
  h0.append(pltpu.async_copy(item_h.at[bsl], it1, s_tab))
  h0.append(pltpu.async_copy(tbin_h.at[bsl], tb1, s_tab))
  h0.append(pltpu.async_copy(tday_h.at[bsl], td_v, s_tab))
  h0.append(pltpu.async_copy(mc_h.at[bsl], mc1, s_tab))
  h0.append(pltpu.async_copy(btd_h, btd_v, s_tab))
  h0.append(pltpu.async_copy(wcu_h, wcu_v, s_tab))
  h0.append(pltpu.async_copy(lr_h, lr_v, s_tab))
  h0.append(pltpu.async_copy(gm_h, gm_v, s_tab))
  for h in h0:
    h.wait()

  for k in range(BPW // 16):
    sl = pl.ds(k * 16, 16)
    u = u1[sl]
    it = it1[sl]
    wb = it * ITEM_BIN + tb1[sl]
    wb1[sl] = wb
    xwbit[sl] = wb >> 4
    ua = u + (u >> 2)
    xua[sl] = ua
    xub[sl] = ua + 1

  hu = []
  hs = []
  for c in range(NCH):
    csl = pl.ds(c * 128, 128)
    hu.append(pltpu.async_copy(uri16_h.at[xua.at[csl]], uriA.at[csl], s_p0))
    hu.append(pltpu.async_copy(uri16_h.at[xub.at[csl]], uriB.at[csl], s_p0))
    uidx = u1.at[csl]
    hs.append(pltpu.async_copy(bu_h.at[uidx], bu_v.at[csl], s_tab))
    hs.append(pltpu.async_copy(mud_h.at[uidx], mud_v.at[csl], s_tab))
    hs.append(pltpu.async_copy(al_h.at[uidx], al_v.at[csl], s_tab))
    hs.append(pltpu.async_copy(bcu_h.at[uidx], bcu_v.at[csl], s_tab))
    hs.append(pltpu.async_copy(uicf_h.at[uidx], uic_v.at[csl], s_tab))
    hs.append(pltpu.async_copy(bi_h.at[it1.at[csl]], bi_v.at[csl], s_tab))

  jobs = []
  for c in range(NCH):
    jobs.append((wbit16_h, xwbit, wb1, wbit_v, c))
  rings = (ring0, ring1, ring2, ring3)
  rsems = (r0, r1, r2, r3)

  def job_copy(j):
    tab, xb, colb, dst, c = jobs[j]
    return pltpu.make_async_copy(tab.at[xb.at[pl.ds(c * 128, 128)]],
                                 rings[j % 4], rsems[j % 4])

  for j in range(len(jobs)):
    job_copy(j).start()
  for j in range(len(jobs)):
    job_copy(j).wait()
    tab, xb, colb, dst, c = jobs[j]
    ring = rings[j % 4]
    for k in range(8):
      sl = pl.ds(c * 128 + k * 16, 16)
      dst[sl] = plsc.load_gather(ring, [k * 16 + iota, colb[sl] & 15])
  for h in hs:
    h.wait()
  for h in hu:
    h.wait()

  def clamp_body(g, _):
    bl = g * 16 + iota
    u16 = u1[pl.ds(pl.multiple_of(g * 16, 16), 16)]
    off = (u16 & 3) * 4
    cnt = jnp.zeros((16,), jnp.float32)
    gslot = g * GSLOT
    for h in range(HIST):
      pos = off + h
      lane = pos & 15
      jA = plsc.load_gather(uriA, [bl, lane])
      jB = plsc.load_gather(uriB, [bl, lane])
      j = jnp.where(pos < 16, jA, jB)
      cnt = cnt + jnp.where(j == N_ITEMS, 1.0, 0.0)
      jc = jnp.minimum(j, N_ITEMS - 1)
      slot = gslot + iota * HIST + h
      plsc.store_scatter(jflat, [slot >> 7, slot & 127], jc)
    cnt_v[pl.ds(pl.multiple_of(g * 16, 16), 16)] = cnt
    for k in range(4):
      pslot = gslot + 320 + k * 16 + iota
      pval = ((g * 64 + k * 16 + iota) * 32 + wid) & 0xFFFF
      plsc.store_scatter(jflat, [pslot >> 7, pslot & 127], pval)
    return 0

  lax.fori_loop(0, NG, clamp_body, 0)

  def g_copies(g, bufs, sem):
    wrow, irow, arow, prow, ybuf = bufs
    gsl = pl.ds(pl.multiple_of(g * 16, 16), 16)
    cps = [
        pltpu.make_async_copy(wpu_h.at[u1.at[gsl]], wrow, sem),
        pltpu.make_async_copy(wpi_h.at[it1.at[gsl]], irow, sem),
        pltpu.make_async_copy(auk_h.at[u1.at[gsl]], arow, sem),
        pltpu.make_async_copy(pkut_h.at[mc1.at[gsl]], prow, sem),
    ]
    for c in range(ROWS_PER_G):
      cps.append(pltpu.make_async_copy(y_h.at[jflat.at[ROWS_PER_G * g + c]],
                                       ybuf.at[pl.ds(c * 128, 128)], sem))
    return cps

  def fire_g(g, bufs, sem):
    for cp in g_copies(g, bufs, sem):
      cp.start()

  def drain_g(g, bufs, sem):
    for cp in g_copies(g, bufs, sem):
      cp.wait()

  bufs0 = (wrow0, irow0, arow0, prow0, ybuf0)
  bufs1 = (wrow1, irow1, arow1, prow1, ybuf1)

  gm16 = gm_v[pl.ds(0, 16)]
  lrA = lr_v[pl.ds(0, 16)]
  lrB = lr_v[pl.ds(16, 16)]

  def compute_group(g, bufs):
    wrow, irow, arow, prow, ybuf = bufs
    sl = pl.ds(pl.multiple_of(g * 16, 16), 16)
    bu = bu_v[sl]
    mud = mud_v[sl]
    al = al_v[sl]
    bcu = bcu_v[sl]
    uic = uic_v[sl]
    bi_ = bi_v[sl]
    wbit = wbit_v[sl]
    cnt = cnt_v[sl]
    td = td_v[sl].astype(jnp.float32)
    mc16 = mc1[sl]
    btd = plsc.load_gather(btd_v, [mc16])
    wcu = plsc.load_gather(wcu_v, [mc16])
    d = td - mud
    dev = jnp.sign(d) * _pow_approx(jnp.abs(d), BETA)
    ru = _pow_approx(uic, -0.5)
    acc = jnp.zeros((16,), jnp.float32)
    for b in range(16):
      h0 = pl.ds(0, 16)
      h1 = pl.ds(16, 16)
      s0 = ((ybuf[b * HIST + 0, h0] + ybuf[b * HIST + 1, h0])
            + (ybuf[b * HIST + 2, h0] + ybuf[b * HIST + 3, h0]))
      s1 = ((ybuf[b * HIST + 0, h1] + ybuf[b * HIST + 1, h1])
            + (ybuf[b * HIST + 2, h1] + ybuf[b * HIST + 3, h1]))
      for h in range(4, HIST, 4):
        s0 = s0 + ((ybuf[b * HIST + h, h0] + ybuf[b * HIST + h + 1, h0])
                   + (ybuf[b * HIST + h + 2, h0] + ybuf[b * HIST + h + 3, h0]))
        s1 = s1 + ((ybuf[b * HIST + h, h1] + ybuf[b * HIST + h + 1, h1])
                   + (ybuf[b * HIST + h + 2, h1] + ybuf[b * HIST + h + 3, h1]))
      cb = cnt[b]
      s0 = s0 - cb * lrA
      s1 = s1 - cb * lrB
      rb = ru[b]
      db = dev[b]
      uvt0 = wrow[b, h0] + rb * s0 + db * arow[b, h0] + prow[b, h0]
      uvt1 = wrow[b, h1] + rb * s1 + db * arow[b, h1] + prow[b, h1]
      prod = uvt0 * irow[b, h0] + uvt1 * irow[b, h1]
      acc = acc + jnp.where(iota == b, jnp.sum(prod), 0.0)
    pred = (gm16 + bu + al * dev + btd
            + (bi_ + wbit) * (bcu + wcu) + acc)
    out_v[sl] = pred

  fire_g(0, bufs0, s_p0)
  fire_g(1, bufs1, s_p1)

  def pipe_body(i, _):
    g0 = i * 2
    g1 = i * 2 + 1
    drain_g(g0, bufs0, s_p0)
    compute_group(g0, bufs0)
    fire_g(jnp.minimum(g0 + 2, NG - 1), bufs0, s_p0)
    drain_g(g1, bufs1, s_p1)
    compute_group(g1, bufs1)
    fire_g(jnp.minimum(g1 + 2, NG - 1), bufs1, s_p1)
    return 0

  lax.fori_loop(0, NG // 2, pipe_body, 0)
  drain_g(NG - 1, bufs0, s_p0)
  drain_g(NG - 1, bufs1, s_p1)

  pltpu.sync_copy(out_v, out_h.at[pl.ds(base, BPW)])


@jax.jit
def _run(user, item, tbin, tday, mc, bu, mud, al, bcu, uicf, bi,
         wbit16, uri16, btd, wcu, wpu, wpi, auk, pkut, y, lr, gm):
  mesh = plsc.VectorSubcoreMesh(core_axis_name="c", subcore_axis_name="s",
                                num_cores=NC, num_subcores=NS)
  f = pl.kernel(
      _body,
      out_type=jax.ShapeDtypeStruct((B,), jnp.float32),
      mesh=mesh,
      scratch_types=[
          pltpu.VMEM((BPW,), jnp.int32),
          pltpu.VMEM((BPW,), jnp.int32),
          pltpu.VMEM((BPW,), jnp.int32),
          pltpu.VMEM((BPW,), jnp.int32),
          pltpu.VMEM((BPW,), jnp.int32),
          pltpu.VMEM((BPW,), jnp.int32),
          pltpu.VMEM((BPW,), jnp.int32),
          pltpu.VMEM((BPW,), jnp.int32),
          pltpu.VMEM((BPW,), jnp.int32),
          pltpu.VMEM((BPW,), jnp.float32),
          pltpu.VMEM((BPW,), jnp.float32),
          pltpu.VMEM((BPW,), jnp.float32),
          pltpu.VMEM((BPW,), jnp.float32),
          pltpu.VMEM((BPW,), jnp.float32),
          pltpu.VMEM((BPW,), jnp.float32),
          pltpu.VMEM((BPW,), jnp.float32),
          pltpu.VMEM((BPW, 16), jnp.int32),
          pltpu.VMEM((BPW, 16), jnp.int32),
          pltpu.VMEM((128, 16), jnp.float32),
          pltpu.VMEM((128, 16), jnp.float32),
          pltpu.VMEM((128, 16), jnp.float32),
          pltpu.VMEM((128, 16), jnp.float32),
          pltpu.VMEM((MAXDAY + 1,), jnp.float32),
          pltpu.VMEM((MAXDAY + 1,), jnp.float32),
          pltpu.VMEM((NG * ROWS_PER_G, 128), jnp.int32),
          pltpu.VMEM((BPW,), jnp.float32),
          pltpu.VMEM((16, N_F), jnp.float32),
          pltpu.VMEM((16, N_F), jnp.float32),
          pltpu.VMEM((16, N_F), jnp.float32),
          pltpu.VMEM((16, N_F), jnp.float32),
          pltpu.VMEM((16, N_F), jnp.float32),
          pltpu.VMEM((16, N_F), jnp.float32),
          pltpu.VMEM((16, N_F), jnp.float32),
          pltpu.VMEM((16, N_F), jnp.float32),
          pltpu.VMEM((GSLOT, N_F), jnp.float32),
          pltpu.VMEM((GSLOT, N_F), jnp.float32),
          pltpu.VMEM((N_F,), jnp.float32),
          pltpu.VMEM((16,), jnp.float32),
          pltpu.VMEM((BPW,), jnp.float32),
          pltpu.SemaphoreType.DMA,
          pltpu.SemaphoreType.DMA,
          pltpu.SemaphoreType.DMA,
          pltpu.SemaphoreType.DMA,
          pltpu.SemaphoreType.DMA,
          pltpu.SemaphoreType.DMA,
          pltpu.SemaphoreType.DMA,
      ],
      compiler_params=pltpu.CompilerParams(needs_layout_passes=False,
                                           use_tc_tiling_on_sc=False),
  )
  return f(user, item, tbin, tday, mc, bu, mud, al, bcu, uicf, bi,
           wbit16, uri16, btd, wcu, wpu, wpi, auk, pkut, y, lr, gm)


def kernel(user, item, tbin, tday, mean_ud, global_mean, maxday_cat,
           user_itemcount, user_rated_item, WPI, WPU, BU, BI, WBIT, Alpha,
           AlphaUK, WPUKT, BTDay, BCU, WCU, Y):
  uicf = user_itemcount.astype(jnp.float32)
  wbit16 = WBIT.reshape(-1, 16)
  uri16 = user_rated_item.reshape(-1, 16)
  lr32 = Y[N_ITEMS - 1]
  gm16 = jnp.broadcast_to(global_mean, (16,)).astype(jnp.float32)
  return _run(user, item, tbin, tday, maxday_cat, BU, mean_ud, Alpha,
              BCU, uicf, BI, wbit16, uri16,
              BTDay, WCU, WPU, WPI, AlphaUK, WPUKT, Y, lr32, gm16)
